# initial kernel scaffold (unmeasured)
import jax
import jax.numpy as jnp
from jax import lax
from jax.experimental import pallas as pl
from jax.experimental.pallas import tpu as pltpu


def kernel(
    x,
):
    def body(*refs):
        pass

    out_shape = jax.ShapeDtypeStruct(..., jnp.float32)
    return pl.pallas_call(body, out_shape=out_shape)(...)


# baseline (device time: 587323 ns/iter reference)
import jax
import jax.numpy as jnp
from jax import lax
from jax.experimental import pallas as pl
from jax.experimental.pallas import tpu as pltpu


def kernel(x):
    m, n = x.shape
    h = m // 2
    w = n // 2

    def body(x_ref, out_ref, xv_ref, rb_ref, send_sems, recv_sems,
             copy_sem, credit_sem):
        my_x = lax.axis_index("x")
        my_y = lax.axis_index("y")
        ox = 1 - my_x
        oy = 1 - my_y

        local = pltpu.make_async_copy(
            x_ref.at[pl.ds(my_x * h, h), :], xv_ref, copy_sem)
        local.start()

        barrier = pltpu.get_barrier_semaphore()
        pl.semaphore_signal(barrier, inc=1, device_id=(ox, my_y),
                            device_id_type=pl.DeviceIdType.MESH)
        pl.semaphore_signal(barrier, inc=1, device_id=(my_x, oy),
                            device_id_type=pl.DeviceIdType.MESH)
        pl.semaphore_wait(barrier, 2)

        p1 = pltpu.make_async_remote_copy(
            src_ref=x_ref.at[pl.ds(ox * h, h), :],
            dst_ref=out_ref.at[pl.ds(my_x * h, h), :],
            send_sem=send_sems.at[0],
            recv_sem=recv_sems.at[0],
            device_id=(ox, my_y),
            device_id_type=pl.DeviceIdType.MESH,
        )
        p1.start()
        local.wait()
        p1.wait()
        out_ref[pl.ds(my_x * h, h), :] = xv_ref[...] + out_ref[pl.ds(ox * h, h), :]
        pl.semaphore_signal(credit_sem, inc=1, device_id=(ox, my_y),
                            device_id_type=pl.DeviceIdType.MESH)

        p2 = pltpu.make_async_remote_copy(
            src_ref=out_ref.at[pl.ds(my_x * h, h), pl.ds(oy * w, w)],
            dst_ref=rb_ref,
            send_sem=send_sems.at[1],
            recv_sem=recv_sems.at[1],
            device_id=(my_x, oy),
            device_id_type=pl.DeviceIdType.MESH,
        )
        p2.start()
        p2.wait()
        out_ref[pl.ds(my_x * h, h), pl.ds(my_y * w, w)] = (
            out_ref[pl.ds(my_x * h, h), pl.ds(my_y * w, w)] + rb_ref[...]
        )

        p3 = pltpu.make_async_remote_copy(
            src_ref=out_ref.at[pl.ds(my_x * h, h), pl.ds(my_y * w, w)],
            dst_ref=out_ref.at[pl.ds(my_x * h, h), pl.ds(my_y * w, w)],
            send_sem=send_sems.at[2],
            recv_sem=recv_sems.at[2],
            device_id=(my_x, oy),
            device_id_type=pl.DeviceIdType.MESH,
        )
        p3.start()
        p3.wait()

        pl.semaphore_wait(credit_sem, 1)
        p4 = pltpu.make_async_remote_copy(
            src_ref=out_ref.at[pl.ds(my_x * h, h), :],
            dst_ref=out_ref.at[pl.ds(my_x * h, h), :],
            send_sem=send_sems.at[3],
            recv_sem=recv_sems.at[3],
            device_id=(ox, my_y),
            device_id_type=pl.DeviceIdType.MESH,
        )
        p4.start()
        p4.wait()

    return pl.pallas_call(
        body,
        out_shape=jax.ShapeDtypeStruct((m, n), jnp.float32),
        in_specs=[pl.BlockSpec(memory_space=pl.ANY)],
        out_specs=pl.BlockSpec(memory_space=pltpu.VMEM),
        scratch_shapes=[
            pltpu.VMEM((h, n), jnp.float32),
            pltpu.VMEM((h, w), jnp.float32),
            pltpu.SemaphoreType.DMA((4,)),
            pltpu.SemaphoreType.DMA((4,)),
            pltpu.SemaphoreType.DMA,
            pltpu.SemaphoreType.REGULAR,
        ],
        compiler_params=pltpu.CompilerParams(
            collective_id=0,
            vmem_limit_bytes=60 * 1024 * 1024,
        ),
    )(x)


# device time: 317946 ns/iter; 1.8472x vs baseline; 1.8472x over previous
import jax
import jax.numpy as jnp
from jax import lax
from jax.experimental import pallas as pl
from jax.experimental.pallas import tpu as pltpu


def kernel(x):
    m, n = x.shape
    h = m // 2
    c = n // 2
    q = n // 4

    def body(x_ref, out_ref, xa_ref, xb_ref, ra_ref, rb_ref,
             send_sems, recv_sems, copy_sems, credit_sems):
        my_x = lax.axis_index("x")
        my_y = lax.axis_index("y")
        ox = 1 - my_x
        oy = 1 - my_y

        def swap(idx, src, dst, dev):
            return pltpu.make_async_remote_copy(
                src_ref=src, dst_ref=dst,
                send_sem=send_sems.at[idx], recv_sem=recv_sems.at[idx],
                device_id=dev, device_id_type=pl.DeviceIdType.MESH,
            )

        x_nbr = (ox, my_y)
        y_nbr = (my_x, oy)

        la = pltpu.make_async_copy(
            x_ref.at[pl.ds(my_x * h, h), pl.ds(0, c)], xa_ref, copy_sems.at[0])
        lb = pltpu.make_async_copy(
            x_ref.at[:, pl.ds(c + my_y * q, q)], xb_ref, copy_sems.at[1])
        la.start()
        lb.start()

        barrier = pltpu.get_barrier_semaphore()
        pl.semaphore_signal(barrier, inc=1, device_id=x_nbr,
                            device_id_type=pl.DeviceIdType.MESH)
        pl.semaphore_signal(barrier, inc=1, device_id=y_nbr,
                            device_id_type=pl.DeviceIdType.MESH)
        pl.semaphore_wait(barrier, 2)

        a1 = swap(0, x_ref.at[pl.ds(ox * h, h), pl.ds(0, c)],
                  out_ref.at[pl.ds(my_x * h, h), pl.ds(0, c)], x_nbr)
        b1 = swap(1, x_ref.at[:, pl.ds(c + oy * q, q)],
                  out_ref.at[:, pl.ds(c + my_y * q, q)], y_nbr)
        a1.start()
        b1.start()
        la.wait()
        a1.wait()
        out_ref[pl.ds(my_x * h, h), pl.ds(0, c)] = (
            xa_ref[...] + out_ref[pl.ds(ox * h, h), pl.ds(0, c)]
        )
        pl.semaphore_signal(credit_sems.at[0], inc=1, device_id=x_nbr,
                            device_id_type=pl.DeviceIdType.MESH)
        lb.wait()
        b1.wait()
        out_ref[:, pl.ds(c + my_y * q, q)] = (
            xb_ref[...] + out_ref[:, pl.ds(c + oy * q, q)]
        )
        pl.semaphore_signal(credit_sems.at[1], inc=1, device_id=y_nbr,
                            device_id_type=pl.DeviceIdType.MESH)

        a2 = swap(2, out_ref.at[pl.ds(my_x * h, h), pl.ds(oy * q, q)],
                  ra_ref, y_nbr)
        b2 = swap(3, out_ref.at[pl.ds(ox * h, h), pl.ds(c + my_y * q, q)],
                  rb_ref, x_nbr)
        a2.start()
        b2.start()
        a2.wait()
        out_ref[pl.ds(my_x * h, h), pl.ds(my_y * q, q)] = (
            out_ref[pl.ds(my_x * h, h), pl.ds(my_y * q, q)] + ra_ref[...]
        )
        b2.wait()
        out_ref[pl.ds(my_x * h, h), pl.ds(c + my_y * q, q)] = (
            out_ref[pl.ds(my_x * h, h), pl.ds(c + my_y * q, q)] + rb_ref[...]
        )

        a3 = swap(4, out_ref.at[pl.ds(my_x * h, h), pl.ds(my_y * q, q)],
                  out_ref.at[pl.ds(my_x * h, h), pl.ds(my_y * q, q)], y_nbr)
        b3 = swap(5, out_ref.at[pl.ds(my_x * h, h), pl.ds(c + my_y * q, q)],
                  out_ref.at[pl.ds(my_x * h, h), pl.ds(c + my_y * q, q)], x_nbr)
        a3.start()
        b3.start()
        a3.wait()
        b3.wait()

        pl.semaphore_wait(credit_sems.at[0], 1)
        a4 = swap(6, out_ref.at[pl.ds(my_x * h, h), pl.ds(0, c)],
                  out_ref.at[pl.ds(my_x * h, h), pl.ds(0, c)], x_nbr)
        a4.start()
        pl.semaphore_wait(credit_sems.at[1], 1)
        b4 = swap(7, out_ref.at[:, pl.ds(c + my_y * q, q)],
                  out_ref.at[:, pl.ds(c + my_y * q, q)], y_nbr)
        b4.start()
        a4.wait()
        b4.wait()

    return pl.pallas_call(
        body,
        out_shape=jax.ShapeDtypeStruct((m, n), jnp.float32),
        in_specs=[pl.BlockSpec(memory_space=pl.ANY)],
        out_specs=pl.BlockSpec(memory_space=pltpu.VMEM),
        scratch_shapes=[
            pltpu.VMEM((h, c), jnp.float32),
            pltpu.VMEM((m, q), jnp.float32),
            pltpu.VMEM((h, q), jnp.float32),
            pltpu.VMEM((h, q), jnp.float32),
            pltpu.SemaphoreType.DMA((8,)),
            pltpu.SemaphoreType.DMA((8,)),
            pltpu.SemaphoreType.DMA((2,)),
            pltpu.SemaphoreType.REGULAR((2,)),
        ],
        compiler_params=pltpu.CompilerParams(
            collective_id=0,
            vmem_limit_bytes=60 * 1024 * 1024,
        ),
    )(x)


# device time: 317187 ns/iter; 1.8517x vs baseline; 1.0024x over previous
import jax
import jax.numpy as jnp
from jax import lax
from jax.experimental import pallas as pl
from jax.experimental.pallas import tpu as pltpu


def kernel(x):
    m, n = x.shape
    h = m // 2
    c = n // 2
    q = n // 4

    def body(x_ref, out_ref, xa_ref, xb_ref, ra_ref, rb_ref,
             send_sems, recv_sems, copy_sems, credit_sems):
        my_x = lax.axis_index("x")
        my_y = lax.axis_index("y")
        ox = 1 - my_x
        oy = 1 - my_y

        def swap(idx, src, dst, dev):
            return pltpu.make_async_remote_copy(
                src_ref=src, dst_ref=dst,
                send_sem=send_sems.at[idx], recv_sem=recv_sems.at[idx],
                device_id=dev, device_id_type=pl.DeviceIdType.MESH,
            )

        x_nbr = (ox, my_y)
        y_nbr = (my_x, oy)

        la = pltpu.make_async_copy(
            x_ref.at[pl.ds(my_x * h, h), pl.ds(0, c)], xa_ref, copy_sems.at[0])
        lb = pltpu.make_async_copy(
            x_ref.at[:, pl.ds(c + my_y * q, q)], xb_ref, copy_sems.at[1])
        la.start()
        lb.start()

        barrier = pltpu.get_barrier_semaphore()
        pl.semaphore_signal(barrier, inc=1, device_id=x_nbr,
                            device_id_type=pl.DeviceIdType.MESH)
        pl.semaphore_signal(barrier, inc=1, device_id=y_nbr,
                            device_id_type=pl.DeviceIdType.MESH)
        pl.semaphore_wait(barrier, 2)

        a1 = swap(0, x_ref.at[pl.ds(ox * h, h), pl.ds(0, c)],
                  out_ref.at[pl.ds(my_x * h, h), pl.ds(0, c)], x_nbr)
        b1 = swap(1, x_ref.at[:, pl.ds(c + oy * q, q)],
                  out_ref.at[:, pl.ds(c + my_y * q, q)], y_nbr)
        a2 = swap(2, out_ref.at[pl.ds(my_x * h, h), pl.ds(oy * q, q)],
                  ra_ref, y_nbr)
        b2 = swap(3, out_ref.at[pl.ds(ox * h, h), pl.ds(c + my_y * q, q)],
                  rb_ref, x_nbr)

        a1.start()
        b1.start()
        la.wait()
        a1.wait()
        out_ref[pl.ds(my_x * h, h), pl.ds(0, c)] = (
            xa_ref[...] + out_ref[pl.ds(ox * h, h), pl.ds(0, c)]
        )
        pl.semaphore_signal(credit_sems.at[0], inc=1, device_id=x_nbr,
                            device_id_type=pl.DeviceIdType.MESH)
        a2.start()
        lb.wait()
        b1.wait()
        out_ref[:, pl.ds(c + my_y * q, q)] = (
            xb_ref[...] + out_ref[:, pl.ds(c + oy * q, q)]
        )
        pl.semaphore_signal(credit_sems.at[1], inc=1, device_id=y_nbr,
                            device_id_type=pl.DeviceIdType.MESH)
        b2.start()

        a3 = swap(4, out_ref.at[pl.ds(my_x * h, h), pl.ds(my_y * q, q)],
                  out_ref.at[pl.ds(my_x * h, h), pl.ds(my_y * q, q)], y_nbr)
        b3 = swap(5, out_ref.at[pl.ds(my_x * h, h), pl.ds(c + my_y * q, q)],
                  out_ref.at[pl.ds(my_x * h, h), pl.ds(c + my_y * q, q)], x_nbr)
        a4 = swap(6, out_ref.at[pl.ds(my_x * h, h), pl.ds(0, c)],
                  out_ref.at[pl.ds(my_x * h, h), pl.ds(0, c)], x_nbr)
        b4 = swap(7, out_ref.at[:, pl.ds(c + my_y * q, q)],
                  out_ref.at[:, pl.ds(c + my_y * q, q)], y_nbr)

        a2.wait()
        out_ref[pl.ds(my_x * h, h), pl.ds(my_y * q, q)] = (
            out_ref[pl.ds(my_x * h, h), pl.ds(my_y * q, q)] + ra_ref[...]
        )
        a3.start()
        b2.wait()
        out_ref[pl.ds(my_x * h, h), pl.ds(c + my_y * q, q)] = (
            out_ref[pl.ds(my_x * h, h), pl.ds(c + my_y * q, q)] + rb_ref[...]
        )
        b3.start()
        a3.wait()
        pl.semaphore_wait(credit_sems.at[0], 1)
        a4.start()
        b3.wait()
        pl.semaphore_wait(credit_sems.at[1], 1)
        b4.start()
        a4.wait()
        b4.wait()

    return pl.pallas_call(
        body,
        out_shape=jax.ShapeDtypeStruct((m, n), jnp.float32),
        in_specs=[pl.BlockSpec(memory_space=pl.ANY)],
        out_specs=pl.BlockSpec(memory_space=pltpu.VMEM),
        scratch_shapes=[
            pltpu.VMEM((h, c), jnp.float32),
            pltpu.VMEM((m, q), jnp.float32),
            pltpu.VMEM((h, q), jnp.float32),
            pltpu.VMEM((h, q), jnp.float32),
            pltpu.SemaphoreType.DMA((8,)),
            pltpu.SemaphoreType.DMA((8,)),
            pltpu.SemaphoreType.DMA((2,)),
            pltpu.SemaphoreType.REGULAR((2,)),
        ],
        compiler_params=pltpu.CompilerParams(
            collective_id=0,
            vmem_limit_bytes=60 * 1024 * 1024,
        ),
    )(x)
